# Initial kernel scaffold; baseline (speedup 1.0000x reference)
import jax
import jax.numpy as jnp
from jax.experimental import pallas as pl


def kernel(x, edge_index, W, att_src, att_dst, bias):
    raise NotImplementedError("write your pallas kernel here")


# speedup vs baseline: 25.2701x; 25.2701x over previous
import jax
import jax.numpy as jnp
from jax import lax
from jax.experimental import pallas as pl
from jax.experimental.pallas import tpu as pltpu
from jax.experimental.pallas import tpu_sc as plsc

N = 10000
E = 320000
D = 128
DP = 144
DCOL = D + 1
NEG_SLOPE = 0.2
NW = 32
EPW = E // NW
CHUNK = 80
NCHUNK = EPW // CHUNK
STAGE = 5
NSTAGE = NCHUNK // STAGE
RPT = N // 16
DRAIN_R = 125
LANES = 16


def _prep_body(x_ref, w_ref, as_ref, ad_ref, h2_ref, adst_ref):
    h = jnp.dot(x_ref[...], w_ref[...],
                preferred_element_type=jnp.float32,
                precision=lax.Precision.HIGHEST)
    a_src = jnp.sum(h * as_ref[0][None, :], axis=1, keepdims=True)
    ones = jnp.ones((N, DP - D - 1), jnp.float32)
    h2_ref[...] = jnp.concatenate([h, a_src, ones], axis=1)
    adst_ref[...] = jnp.sum(h * ad_ref[0][None, :], axis=1, keepdims=True)


_tc_prep = pl.pallas_call(
    _prep_body,
    out_shape=[
        jax.ShapeDtypeStruct((N, DP), jnp.float32),
        jax.ShapeDtypeStruct((N, 1), jnp.float32),
    ],
)


def _final_body(p_ref, b_ref, o_ref):
    s = p_ref[0] + p_ref[1]
    denom = s[:, DCOL:DCOL + 1]
    o_ref[...] = s[:, :D] / (denom + 1e-16) + b_ref[...]


_tc_final = pl.pallas_call(
    _final_body,
    out_shape=jax.ShapeDtypeStruct((N, D), jnp.float32),
)


def _sc_edge_body(adst_hbm, src_hbm, dst_hbm, h2_hbm, out_hbm,
                  adst_v, sstage_v, dstage_v, rows_v, ex_v, acc_sh):
    cid = lax.axis_index("c")
    sid = lax.axis_index("s")
    wid = cid * 16 + sid

    pltpu.sync_copy(adst_hbm, adst_v)

    @pl.loop(0, DRAIN_R)
    def _(r):
        for k in range(DP // LANES):
            rows_v[r, pl.ds(k * LANES, LANES)] = jnp.zeros((LANES,), jnp.float32)

    @pl.loop(0, RPT // DRAIN_R)
    def _(b):
        pltpu.sync_copy(rows_v, acc_sh.at[pl.ds(sid * RPT + b * DRAIN_R, DRAIN_R)])

    plsc.subcore_barrier()

    col = jnp.full((LANES,), D, jnp.int32)

    @pl.loop(0, NSTAGE)
    def _(cc):
        pltpu.sync_copy(src_hbm.at[wid, pl.ds(cc * STAGE, STAGE)], sstage_v)
        pltpu.sync_copy(dst_hbm.at[wid, pl.ds(cc * STAGE, STAGE)], dstage_v)
        for b in range(STAGE):
            pltpu.sync_copy(h2_hbm.at[sstage_v.at[b]], rows_v.at[pl.ds(0, CHUNK)])
            for k in range(CHUNK // LANES):
                rowsel = lax.iota(jnp.int32, LANES) + (k * LANES)
                asg = plsc.load_gather(rows_v, [rowsel, col])
                dv = dstage_v[b, pl.ds(k * LANES, LANES)]
                adg = plsc.load_gather(adst_v, [dv])
                e = asg + adg
                e = jnp.where(e < 0.0, e * NEG_SLOPE, e)
                ex_v[pl.ds(k * LANES, LANES)] = jnp.exp(e)

            @pl.loop(0, CHUNK)
            def _(j):
                al = plsc.load_gather(ex_v, [jnp.zeros((LANES,), jnp.int32) + j])
                for r in range(DP // LANES):
                    rows_v[j, pl.ds(r * LANES, LANES)] = (
                        rows_v[j, pl.ds(r * LANES, LANES)] * al)

            pltpu.sync_copy(rows_v.at[pl.ds(0, CHUNK)],
                            acc_sh.at[dstage_v.at[b]], add=True)

    plsc.subcore_barrier()

    @pl.loop(0, RPT // DRAIN_R)
    def _(b):
        sl = pl.ds(sid * RPT + b * DRAIN_R, DRAIN_R)
        pltpu.sync_copy(acc_sh.at[sl], rows_v)
        pltpu.sync_copy(rows_v, out_hbm.at[cid, sl])


_sc_edge = pl.kernel(
    _sc_edge_body,
    out_type=jax.ShapeDtypeStruct((2, N, DP), jnp.float32),
    mesh=plsc.VectorSubcoreMesh(core_axis_name="c", subcore_axis_name="s"),
    compiler_params=pltpu.CompilerParams(use_tc_tiling_on_sc=False,
                                         needs_layout_passes=False),
    scratch_types=[
        pltpu.VMEM((N,), jnp.float32),
        pltpu.VMEM((STAGE, CHUNK), jnp.int32),
        pltpu.VMEM((STAGE, CHUNK), jnp.int32),
        pltpu.VMEM((DRAIN_R, DP), jnp.float32),
        pltpu.VMEM((CHUNK,), jnp.float32),
        pltpu.VMEM_SHARED((N, DP), jnp.float32),
    ],
)


def kernel(x, edge_index, W, att_src, att_dst, bias):
    src3 = edge_index[0].reshape(NW, NCHUNK, CHUNK)
    dst3 = edge_index[1].reshape(NW, NCHUNK, CHUNK)
    h2, a_dst = _tc_prep(x, W, att_src, att_dst)
    parts = _sc_edge(a_dst.reshape(N), src3, dst3, h2)
    return _tc_final(parts, bias.reshape(1, D))


# speedup vs baseline: 38.9276x; 1.5405x over previous
import jax
import jax.numpy as jnp
from jax import lax
from jax.experimental import pallas as pl
from jax.experimental.pallas import tpu as pltpu
from jax.experimental.pallas import tpu_sc as plsc

N = 10000
E = 320000
D = 128
DP = 144
DCOL = D + 1
NEG_SLOPE = 0.2
NW = 32
EPW = E // NW
CHUNK = 80
NCHUNK = EPW // CHUNK
WIN = 25
NWIN = NCHUNK // WIN
RPT = N // 16
LANES = 16
NBUF = 3


def _prep_body(x_ref, w_ref, as_ref, ad_ref, h2_ref, adst_ref):
    h = jnp.dot(x_ref[...], w_ref[...],
                preferred_element_type=jnp.float32,
                precision=lax.Precision.HIGHEST)
    a_src = jnp.sum(h * as_ref[0][None, :], axis=1, keepdims=True)
    ones = jnp.ones((N, DP - D - 1), jnp.float32)
    h2_ref[...] = jnp.concatenate([h, a_src, ones], axis=1)
    adst_ref[...] = jnp.sum(h * ad_ref[0][None, :], axis=1, keepdims=True)


_tc_prep = pl.pallas_call(
    _prep_body,
    out_shape=[
        jax.ShapeDtypeStruct((N, DP), jnp.float32),
        jax.ShapeDtypeStruct((N, 1), jnp.float32),
    ],
)


def _final_body(p_ref, b_ref, o_ref):
    s = p_ref[0] + p_ref[1]
    denom = s[:, DCOL:DCOL + 1]
    o_ref[...] = s[:, :D] / (denom + 1e-16) + b_ref[...]


_tc_final = pl.pallas_call(
    _final_body,
    out_shape=jax.ShapeDtypeStruct((N, D), jnp.float32),
)


def _sc_edge_body(adst_hbm, sd_hbm, h2_hbm, zero_hbm, out_hbm,
                  sdw_v, rows_v, adv_v, ex_v, acc_sh,
                  gsems, asems, ssems, wsem):
    cid = lax.axis_index("c")
    sid = lax.axis_index("s")
    wid = cid * 16 + sid

    pltpu.sync_copy(zero_hbm, acc_sh.at[pl.ds(sid * RPT, RPT)])
    plsc.subcore_barrier()

    col = jnp.full((LANES,), D, jnp.int32)

    def process(c, p):
        for k in range(CHUNK // LANES):
            rowsel = lax.iota(jnp.int32, LANES) + (k * LANES)
            asg = plsc.load_gather(rows_v[p], [rowsel, col])
            adg = adv_v[p][pl.ds(k * LANES, LANES)]
            e = asg + adg
            e = jnp.where(e < 0.0, e * NEG_SLOPE, e)
            ex_v[pl.ds(k * LANES, LANES)] = jnp.exp(e)

        @pl.loop(0, CHUNK)
        def _(j):
            al = plsc.load_gather(ex_v, [jnp.zeros((LANES,), jnp.int32) + j])
            for r in range(DP // LANES):
                rows_v[p][j, pl.ds(r * LANES, LANES)] = (
                    rows_v[p][j, pl.ds(r * LANES, LANES)] * al)

    @pl.loop(0, NWIN)
    def _(w):
        pltpu.sync_copy(sd_hbm.at[wid, pl.ds(w * WIN, WIN)], sdw_v)

        def gather(c, p):
            g = pltpu.async_copy(h2_hbm.at[sdw_v.at[c, 0]], rows_v[p], gsems.at[p])
            a = pltpu.async_copy(adst_hbm.at[sdw_v.at[c, 1]], adv_v[p], asems.at[p])
            return g, a

        gh = [None] * NBUF
        sh = [None] * NBUF
        gh[0] = gather(0, 0)
        for c in range(WIN):
            p = c % NBUF
            q = (c + 1) % NBUF
            if c + 1 < WIN:
                if sh[q] is not None:
                    sh[q].wait()
                    sh[q] = None
                gh[q] = gather(c + 1, q)
            g, a = gh[p]
            g.wait()
            a.wait()
            process(c, p)
            sh[p] = pltpu.async_copy(rows_v[p], acc_sh.at[sdw_v.at[c, 1]],
                                     ssems.at[p], add=True)
        for p in range(NBUF):
            if sh[p] is not None:
                sh[p].wait()

    plsc.subcore_barrier()

    pltpu.async_copy(acc_sh.at[pl.ds(sid * RPT, RPT)],
                     out_hbm.at[cid, pl.ds(sid * RPT, RPT)], wsem).wait()


_sc_edge = pl.kernel(
    _sc_edge_body,
    out_type=jax.ShapeDtypeStruct((2, N, DP), jnp.float32),
    mesh=plsc.VectorSubcoreMesh(core_axis_name="c", subcore_axis_name="s"),
    compiler_params=pltpu.CompilerParams(use_tc_tiling_on_sc=False,
                                         needs_layout_passes=False),
    scratch_types=[
        pltpu.VMEM((WIN, 2, CHUNK), jnp.int32),
        [pltpu.VMEM((CHUNK, DP), jnp.float32)] * NBUF,
        [pltpu.VMEM((CHUNK,), jnp.float32)] * NBUF,
        pltpu.VMEM((CHUNK,), jnp.float32),
        pltpu.VMEM_SHARED((N, DP), jnp.float32),
        pltpu.SemaphoreType.DMA((NBUF,)),
        pltpu.SemaphoreType.DMA((NBUF,)),
        pltpu.SemaphoreType.DMA((NBUF,)),
        pltpu.SemaphoreType.DMA,
    ],
)


def kernel(x, edge_index, W, att_src, att_dst, bias):
    sd = jnp.stack([edge_index[0].reshape(NW, NCHUNK, CHUNK),
                    edge_index[1].reshape(NW, NCHUNK, CHUNK)], axis=2)
    zero = jnp.zeros((RPT, DP), jnp.float32)
    h2, a_dst = _tc_prep(x, W, att_src, att_dst)
    parts = _sc_edge(a_dst.reshape(N), sd, h2, zero)
    return _tc_final(parts, bias.reshape(1, D))


# speedup vs baseline: 46.8037x; 1.2023x over previous
import jax
import jax.numpy as jnp
from jax import lax
from jax.experimental import pallas as pl
from jax.experimental.pallas import tpu as pltpu
from jax.experimental.pallas import tpu_sc as plsc

N = 10000
E = 320000
D = 128
DP = 144
DCOL = D + 1
NEG_SLOPE = 0.2
NW = 32
EPW = E // NW
CHUNK = 80
NCHUNK = EPW // CHUNK
WIN = 25
NWIN = NCHUNK // WIN
RPT = N // 16
LANES = 16
NBUF = 3


def _prep_body(x_ref, w_ref, as_ref, ad_ref, h2_ref, adst_ref):
    h = jnp.dot(x_ref[...], w_ref[...],
                preferred_element_type=jnp.float32,
                precision=lax.Precision.HIGHEST)
    a_src = jnp.sum(h * as_ref[0][None, :], axis=1, keepdims=True)
    ones = jnp.ones((N, DP - D - 1), jnp.float32)
    h2_ref[...] = jnp.concatenate([h, a_src, ones], axis=1)
    adst_ref[...] = jnp.sum(h * ad_ref[0][None, :], axis=1, keepdims=True)


_tc_prep = pl.pallas_call(
    _prep_body,
    out_shape=[
        jax.ShapeDtypeStruct((N, DP), jnp.float32),
        jax.ShapeDtypeStruct((N, 1), jnp.float32),
    ],
)


def _final_body(p_ref, b_ref, o_ref):
    s = p_ref[0] + p_ref[1]
    denom = s[:, DCOL:DCOL + 1]
    o_ref[...] = s[:, :D] / (denom + 1e-16) + b_ref[...]


_tc_final = pl.pallas_call(
    _final_body,
    out_shape=jax.ShapeDtypeStruct((N, D), jnp.float32),
)


def _sc_edge_body(adst_hbm, ei_hbm, h2_hbm, zero_hbm, out_hbm,
                  srcw_v, dstw_v, rows_v, adv_v, ex_v, acc_sh,
                  gsems, asems, ssems, wsem):
    cid = lax.axis_index("c")
    sid = lax.axis_index("s")
    wid = cid * 16 + sid

    pltpu.sync_copy(zero_hbm, acc_sh.at[pl.ds(sid * RPT, RPT)])
    plsc.subcore_barrier()

    col = jnp.full((LANES,), D, jnp.int32)

    def process(c, p):
        for k in range(CHUNK // LANES):
            rowsel = lax.iota(jnp.int32, LANES) + (k * LANES)
            asg = plsc.load_gather(rows_v[p], [rowsel, col])
            adg = adv_v[p][pl.ds(k * LANES, LANES)]
            e = asg + adg
            e = jnp.where(e < 0.0, e * NEG_SLOPE, e)
            ex_v[pl.ds(k * LANES, LANES)] = jnp.exp(e)

        @plsc.parallel_loop(0, CHUNK, unroll=4)
        def _(j):
            al = plsc.load_gather(ex_v, [jnp.zeros((LANES,), jnp.int32) + j])
            for r in range(DP // LANES):
                rows_v[p][j, pl.ds(r * LANES, LANES)] = (
                    rows_v[p][j, pl.ds(r * LANES, LANES)] * al)

    @pl.loop(0, NWIN)
    def _(w):
        base = wid * EPW + w * (WIN * CHUNK)
        pltpu.sync_copy(ei_hbm.at[0, pl.ds(base, WIN * CHUNK)], srcw_v)
        pltpu.sync_copy(ei_hbm.at[1, pl.ds(base, WIN * CHUNK)], dstw_v)

        def gather(c, p):
            g = pltpu.async_copy(h2_hbm.at[srcw_v.at[pl.ds(c * CHUNK, CHUNK)]],
                                 rows_v[p], gsems.at[p])
            a = pltpu.async_copy(adst_hbm.at[dstw_v.at[pl.ds(c * CHUNK, CHUNK)]],
                                 adv_v[p], asems.at[p])
            return g, a

        gh = [None] * NBUF
        sh = [None] * NBUF
        gh[0] = gather(0, 0)
        for c in range(WIN):
            p = c % NBUF
            q = (c + 1) % NBUF
            if c + 1 < WIN:
                if sh[q] is not None:
                    sh[q].wait()
                    sh[q] = None
                gh[q] = gather(c + 1, q)
            g, a = gh[p]
            g.wait()
            a.wait()
            process(c, p)
            sh[p] = pltpu.async_copy(rows_v[p],
                                     acc_sh.at[dstw_v.at[pl.ds(c * CHUNK, CHUNK)]],
                                     ssems.at[p], add=True)
        for p in range(NBUF):
            if sh[p] is not None:
                sh[p].wait()

    plsc.subcore_barrier()

    pltpu.async_copy(acc_sh.at[pl.ds(sid * RPT, RPT)],
                     out_hbm.at[cid, pl.ds(sid * RPT, RPT)], wsem).wait()


_sc_edge = pl.kernel(
    _sc_edge_body,
    out_type=jax.ShapeDtypeStruct((2, N, DP), jnp.float32),
    mesh=plsc.VectorSubcoreMesh(core_axis_name="c", subcore_axis_name="s"),
    compiler_params=pltpu.CompilerParams(use_tc_tiling_on_sc=False,
                                         needs_layout_passes=False),
    scratch_types=[
        pltpu.VMEM((WIN * CHUNK,), jnp.int32),
        pltpu.VMEM((WIN * CHUNK,), jnp.int32),
        [pltpu.VMEM((CHUNK, DP), jnp.float32)] * NBUF,
        [pltpu.VMEM((CHUNK,), jnp.float32)] * NBUF,
        pltpu.VMEM((CHUNK,), jnp.float32),
        pltpu.VMEM_SHARED((N, DP), jnp.float32),
        pltpu.SemaphoreType.DMA((NBUF,)),
        pltpu.SemaphoreType.DMA((NBUF,)),
        pltpu.SemaphoreType.DMA((NBUF,)),
        pltpu.SemaphoreType.DMA,
    ],
)


def kernel(x, edge_index, W, att_src, att_dst, bias):
    zero = jnp.zeros((RPT, DP), jnp.float32)
    h2, a_dst = _tc_prep(x, W, att_src, att_dst)
    parts = _sc_edge(a_dst.reshape(N), edge_index, h2, zero)
    return _tc_final(parts, bias.reshape(1, D))
